# baseline (speedup 1.0000x reference)
import jax
import jax.numpy as jnp
from jax.experimental import pallas as pl

B = 4096
F = 4096
BR = 256
NB = B // BR


def _body(x_ref, fs_ref, out_ref):
    x = x_ref[...]
    m = jnp.max(x, axis=-1, keepdims=True)
    iota = jax.lax.broadcasted_iota(jnp.int32, (BR, F), 1)
    masked = jnp.where(x == m, iota, F)
    idx = jnp.min(masked, axis=-1, keepdims=True)
    onehot = (iota == idx)
    fs = fs_ref[...]
    picked = jnp.where(onehot, fs, jnp.float32(0.0))
    out_ref[...] = jnp.sum(picked, axis=-1).reshape(1, 1, BR)


def kernel(input, fspace):
    fs2 = fspace.reshape(1, F)
    nfeat = input.shape[1]
    flat = input.reshape(B, nfeat * F)
    out = pl.pallas_call(
        _body,
        grid=(NB,),
        in_specs=[
            pl.BlockSpec((BR, F), lambda i: (i, 0)),
            pl.BlockSpec((1, F), lambda i: (0, 0)),
        ],
        out_specs=pl.BlockSpec((1, 1, BR), lambda i: (i, 0, 0)),
        out_shape=jax.ShapeDtypeStruct((NB, 1, BR), jnp.float32),
    )(flat, fs2)
    return out.reshape(B)
